# baseline (speedup 1.0000x reference)
import jax
import jax.numpy as jnp
from jax.experimental import pallas as pl
from jax.experimental.pallas import tpu as pltpu

B = 8
C = 1024
HW = 64 * 64
KEY_DIM = 256
MAX_REFS = 8

ROWS = B * HW
R_BLK = 512
N_BLKS = ROWS // R_BLK
BLKS_PER_B = HW // R_BLK


def _stream_body(x_ref, out_ref, psum_ref):
    blk = x_ref[...]
    out_ref[...] = blk
    psum_ref[0] = jnp.sum(blk, axis=0, keepdims=True)


def _retrieval_body(y_ref, psum_ref, w_ref, b_ref, keys_ref, mask_ref,
                    kf_ref, out_ref):
    means = jnp.sum(psum_ref[...], axis=1) * (1.0 / HW)
    q = jax.lax.dot_general(
        means, w_ref[...], (((1,), (1,)), ((), ())),
        preferred_element_type=jnp.float32,
    ) + b_ref[...]
    qn = q / jnp.maximum(
        jnp.sqrt(jnp.sum(q * q, axis=1, keepdims=True)), 1e-12)
    keys = keys_ref[...]
    kn = keys / jnp.maximum(
        jnp.sqrt(jnp.sum(keys * keys, axis=1, keepdims=True)), 1e-12)
    sims = jax.lax.dot_general(
        qn, kn, (((1,), (1,)), ((), ())),
        preferred_element_type=jnp.float32,
    )
    maskf = mask_ref[...]
    masked = jnp.where(maskf > 0.0, sims, -1e30)
    m1 = jnp.max(masked, axis=1, keepdims=True)
    idx = jax.lax.broadcasted_iota(jnp.int32, (B, MAX_REFS), 1)
    pos = jnp.min(jnp.where(masked == m1, idx, MAX_REFS), axis=1,
                  keepdims=True)
    m2 = jnp.max(jnp.where(idx == pos, -3e38, masked), axis=1, keepdims=True)
    e = jnp.exp((m2 - m1) * 10.0)
    denom = 1.0 + e
    attn_sum = jnp.sum(1.0 / denom + e / denom)
    valid = jnp.sum(maskf) * (1.0 / B)
    anchor = 0.0 * (attn_sum + kf_ref[0, 0] + valid)
    out_ref[...] = y_ref[...] + anchor


def kernel(current_context, k, memory_keys, memory_initialized,
           query_proj_w, query_proj_b):
    x = jnp.transpose(current_context, (0, 2, 3, 1)).reshape(ROWS, C)
    kf = jnp.asarray(k, jnp.float32).reshape(1, 1)
    keys = memory_keys[0]
    maskf = jnp.broadcast_to(
        memory_initialized.astype(jnp.float32)[None, :], (B, MAX_REFS))
    bias = query_proj_b.reshape(1, KEY_DIM)

    y, psums = pl.pallas_call(
        _stream_body,
        grid=(N_BLKS,),
        in_specs=[pl.BlockSpec((R_BLK, C), lambda i: (i, 0))],
        out_specs=[
            pl.BlockSpec((R_BLK, C), lambda i: (i, 0)),
            pl.BlockSpec((1, 1, C), lambda i: (i, 0, 0)),
        ],
        out_shape=[
            jax.ShapeDtypeStruct((ROWS, C), jnp.float32),
            jax.ShapeDtypeStruct((N_BLKS, 1, C), jnp.float32),
        ],
    )(x)

    psums2 = psums.reshape(B, BLKS_PER_B, C)

    out = pl.pallas_call(
        _retrieval_body,
        grid=(1,),
        in_specs=[
            pl.BlockSpec((8, C), lambda i: (0, 0)),
            pl.BlockSpec((B, BLKS_PER_B, C), lambda i: (0, 0, 0)),
            pl.BlockSpec((KEY_DIM, C), lambda i: (0, 0)),
            pl.BlockSpec((1, KEY_DIM), lambda i: (0, 0)),
            pl.BlockSpec((MAX_REFS, KEY_DIM), lambda i: (0, 0)),
            pl.BlockSpec((B, MAX_REFS), lambda i: (0, 0)),
            pl.BlockSpec(memory_space=pltpu.SMEM),
        ],
        out_specs=pl.BlockSpec((8, C), lambda i: (0, 0)),
        out_shape=jax.ShapeDtypeStruct((ROWS, C), jnp.float32),
        input_output_aliases={0: 0},
    )(y, psums2, query_proj_w, bias, keys, maskf, kf)
    return jnp.transpose(out.reshape(B, 64, 64, C), (0, 3, 1, 2))


# speedup vs baseline: 1.0452x; 1.0452x over previous
import jax
import jax.numpy as jnp
from jax.experimental import pallas as pl
from jax.experimental.pallas import tpu as pltpu

B = 8
C = 1024
HW = 64 * 64
KEY_DIM = 256
MAX_REFS = 8

ROWS = B * HW
R_BLK = 1024
N_BLKS = ROWS // R_BLK
BLKS_PER_B = HW // R_BLK


def _stream_body(x_ref, out_ref, psum_ref):
    blk = x_ref[...]
    out_ref[...] = blk
    psum_ref[0] = jnp.sum(blk, axis=0, keepdims=True)


def _retrieval_body(y_ref, psum_ref, w_ref, b_ref, keys_ref, mask_ref,
                    kf_ref, out_ref):
    means = jnp.sum(psum_ref[...], axis=1) * (1.0 / HW)
    q = jax.lax.dot_general(
        means, w_ref[...], (((1,), (1,)), ((), ())),
        preferred_element_type=jnp.float32,
    ) + b_ref[...]
    qn = q / jnp.maximum(
        jnp.sqrt(jnp.sum(q * q, axis=1, keepdims=True)), 1e-12)
    keys = keys_ref[...]
    kn = keys / jnp.maximum(
        jnp.sqrt(jnp.sum(keys * keys, axis=1, keepdims=True)), 1e-12)
    sims = jax.lax.dot_general(
        qn, kn, (((1,), (1,)), ((), ())),
        preferred_element_type=jnp.float32,
    )
    maskf = mask_ref[...]
    masked = jnp.where(maskf > 0.0, sims, -1e30)
    m1 = jnp.max(masked, axis=1, keepdims=True)
    idx = jax.lax.broadcasted_iota(jnp.int32, (B, MAX_REFS), 1)
    pos = jnp.min(jnp.where(masked == m1, idx, MAX_REFS), axis=1,
                  keepdims=True)
    m2 = jnp.max(jnp.where(idx == pos, -3e38, masked), axis=1, keepdims=True)
    e = jnp.exp((m2 - m1) * 10.0)
    denom = 1.0 + e
    attn_sum = jnp.sum(1.0 / denom + e / denom)
    valid = jnp.sum(maskf) * (1.0 / B)
    anchor = 0.0 * (attn_sum + kf_ref[0, 0] + valid)
    out_ref[...] = y_ref[...] + anchor


def kernel(current_context, k, memory_keys, memory_initialized,
           query_proj_w, query_proj_b):
    x = jnp.transpose(current_context, (0, 2, 3, 1)).reshape(ROWS, C)
    kf = jnp.asarray(k, jnp.float32).reshape(1, 1)
    keys = memory_keys[0]
    maskf = jnp.broadcast_to(
        memory_initialized.astype(jnp.float32)[None, :], (B, MAX_REFS))
    bias = query_proj_b.reshape(1, KEY_DIM)

    y, psums = pl.pallas_call(
        _stream_body,
        grid=(N_BLKS,),
        in_specs=[pl.BlockSpec((R_BLK, C), lambda i: (i, 0))],
        out_specs=[
            pl.BlockSpec((R_BLK, C), lambda i: (i, 0)),
            pl.BlockSpec((1, 1, C), lambda i: (i, 0, 0)),
        ],
        out_shape=[
            jax.ShapeDtypeStruct((ROWS, C), jnp.float32),
            jax.ShapeDtypeStruct((N_BLKS, 1, C), jnp.float32),
        ],
    )(x)

    psums2 = psums.reshape(B, BLKS_PER_B, C)

    out = pl.pallas_call(
        _retrieval_body,
        grid=(1,),
        in_specs=[
            pl.BlockSpec((8, C), lambda i: (0, 0)),
            pl.BlockSpec((B, BLKS_PER_B, C), lambda i: (0, 0, 0)),
            pl.BlockSpec((KEY_DIM, C), lambda i: (0, 0)),
            pl.BlockSpec((1, KEY_DIM), lambda i: (0, 0)),
            pl.BlockSpec((MAX_REFS, KEY_DIM), lambda i: (0, 0)),
            pl.BlockSpec((B, MAX_REFS), lambda i: (0, 0)),
            pl.BlockSpec(memory_space=pltpu.SMEM),
        ],
        out_specs=pl.BlockSpec((8, C), lambda i: (0, 0)),
        out_shape=jax.ShapeDtypeStruct((ROWS, C), jnp.float32),
        input_output_aliases={0: 0},
    )(y, psums2, query_proj_w, bias, keys, maskf, kf)
    return jnp.transpose(out.reshape(B, 64, 64, C), (0, 3, 1, 2))


# speedup vs baseline: 1.0674x; 1.0212x over previous
import jax
import jax.numpy as jnp
from jax.experimental import pallas as pl
from jax.experimental.pallas import tpu as pltpu

B = 8
C = 1024
HW = 64 * 64
KEY_DIM = 256
MAX_REFS = 8

ROWS = B * HW
R_BLK = 2048
N_BLKS = ROWS // R_BLK
BLKS_PER_B = HW // R_BLK


def _stream_body(x_ref, out_ref, psum_ref):
    blk = x_ref[...]
    out_ref[...] = blk
    psum_ref[0] = jnp.sum(blk, axis=0, keepdims=True)


def _retrieval_body(y_ref, psum_ref, w_ref, b_ref, keys_ref, mask_ref,
                    kf_ref, out_ref):
    means = jnp.sum(psum_ref[...], axis=1) * (1.0 / HW)
    q = jax.lax.dot_general(
        means, w_ref[...], (((1,), (1,)), ((), ())),
        preferred_element_type=jnp.float32,
    ) + b_ref[...]
    qn = q / jnp.maximum(
        jnp.sqrt(jnp.sum(q * q, axis=1, keepdims=True)), 1e-12)
    keys = keys_ref[...]
    kn = keys / jnp.maximum(
        jnp.sqrt(jnp.sum(keys * keys, axis=1, keepdims=True)), 1e-12)
    sims = jax.lax.dot_general(
        qn, kn, (((1,), (1,)), ((), ())),
        preferred_element_type=jnp.float32,
    )
    maskf = mask_ref[...]
    masked = jnp.where(maskf > 0.0, sims, -1e30)
    m1 = jnp.max(masked, axis=1, keepdims=True)
    idx = jax.lax.broadcasted_iota(jnp.int32, (B, MAX_REFS), 1)
    pos = jnp.min(jnp.where(masked == m1, idx, MAX_REFS), axis=1,
                  keepdims=True)
    m2 = jnp.max(jnp.where(idx == pos, -3e38, masked), axis=1, keepdims=True)
    e = jnp.exp((m2 - m1) * 10.0)
    denom = 1.0 + e
    attn_sum = jnp.sum(1.0 / denom + e / denom)
    valid = jnp.sum(maskf) * (1.0 / B)
    anchor = 0.0 * (attn_sum + kf_ref[0, 0] + valid)
    out_ref[...] = y_ref[...] + anchor


def kernel(current_context, k, memory_keys, memory_initialized,
           query_proj_w, query_proj_b):
    x = jnp.transpose(current_context, (0, 2, 3, 1)).reshape(ROWS, C)
    kf = jnp.asarray(k, jnp.float32).reshape(1, 1)
    keys = memory_keys[0]
    maskf = jnp.broadcast_to(
        memory_initialized.astype(jnp.float32)[None, :], (B, MAX_REFS))
    bias = query_proj_b.reshape(1, KEY_DIM)

    y, psums = pl.pallas_call(
        _stream_body,
        grid=(N_BLKS,),
        in_specs=[pl.BlockSpec((R_BLK, C), lambda i: (i, 0))],
        out_specs=[
            pl.BlockSpec((R_BLK, C), lambda i: (i, 0)),
            pl.BlockSpec((1, 1, C), lambda i: (i, 0, 0)),
        ],
        out_shape=[
            jax.ShapeDtypeStruct((ROWS, C), jnp.float32),
            jax.ShapeDtypeStruct((N_BLKS, 1, C), jnp.float32),
        ],
    )(x)

    psums2 = psums.reshape(B, BLKS_PER_B, C)

    out = pl.pallas_call(
        _retrieval_body,
        grid=(1,),
        in_specs=[
            pl.BlockSpec((8, C), lambda i: (0, 0)),
            pl.BlockSpec((B, BLKS_PER_B, C), lambda i: (0, 0, 0)),
            pl.BlockSpec((KEY_DIM, C), lambda i: (0, 0)),
            pl.BlockSpec((1, KEY_DIM), lambda i: (0, 0)),
            pl.BlockSpec((MAX_REFS, KEY_DIM), lambda i: (0, 0)),
            pl.BlockSpec((B, MAX_REFS), lambda i: (0, 0)),
            pl.BlockSpec(memory_space=pltpu.SMEM),
        ],
        out_specs=pl.BlockSpec((8, C), lambda i: (0, 0)),
        out_shape=jax.ShapeDtypeStruct((ROWS, C), jnp.float32),
        input_output_aliases={0: 0},
    )(y, psums2, query_proj_w, bias, keys, maskf, kf)
    return jnp.transpose(out.reshape(B, 64, 64, C), (0, 3, 1, 2))
